# baseline (speedup 1.0000x reference)
import functools

import jax
import jax.numpy as jnp
from jax import lax
from jax.experimental import pallas as pl
from jax.experimental.pallas import tpu as pltpu
from jax.experimental.pallas import tpu_sc as plsc

_B, _N, _K = 4, 5, 3
_C, _H, _W = 256, 96, 96
_CHW = _C * _H * _W
_CL = 3072
_SPLIT = _CHW // _CL
_ROWS_OUT = _B * _K
_TOTAL_CHUNKS = _ROWS_OUT * _SPLIT
_NC, _NS = 2, 16
_NW = _NC * _NS
_TPR = 8
_TASKS = _ROWS_OUT * _TPR
_TPW = _TASKS // _NW
_CPT = _SPLIT // _TPR
_TROWS = _B * _K * _K

_mesh = plsc.VectorSubcoreMesh(
    core_axis_name="c", subcore_axis_name="s", num_cores=_NC, num_subcores=_NS
)


@functools.partial(
    pl.kernel,
    out_type=(
        jax.ShapeDtypeStruct((_TOTAL_CHUNKS, _CL), jnp.float32),
        jax.ShapeDtypeStruct((_TROWS, 16), jnp.float32),
    ),
    mesh=_mesh,
    compiler_params=pltpu.CompilerParams(needs_layout_passes=False),
    scratch_types=[
        pltpu.VMEM((16,), jnp.int32),
        pltpu.VMEM((16,), jnp.int32),
        pltpu.VMEM((_B * _N * _N, 16), jnp.float32),
        pltpu.VMEM((_TROWS, 16), jnp.float32),
        pltpu.SemaphoreType.DMA,
    ],
)
def _sc_gather(feat_ref, t_ref, keep_ref, rl_ref, out_ref, outt_ref,
               keep_v, rl_v, t_v, tbuf, sem):
    wid = lax.axis_index("s") * _NC + lax.axis_index("c")
    pltpu.sync_copy(keep_ref, keep_v)
    pltpu.sync_copy(rl_ref, rl_v)

    lane = lax.iota(jnp.int32, 16)
    keep = keep_v[...]
    rl = rl_v[...]
    grow = keep
    for t in range(_B):
        rl_t = jnp.take_along_axis(rl, jnp.full((16,), t, jnp.int32), axis=0)
        grow = grow + jnp.where(lane // _K > t, rl_t, 0)

    copies = []
    for j in range(_TPW):
        tsk = wid * _TPW + j
        row = tsk // _TPR
        piece = tsk % _TPR
        g = jnp.max(jnp.where(lane == row, grow, 0))
        src = g * _SPLIT + piece * _CPT
        copies.append(pltpu.async_copy(
            feat_ref.at[pl.ds(src, _CPT)],
            out_ref.at[pl.ds(tsk * _CPT, _CPT)], sem))
    for c in copies:
        c.wait()

    @pl.when(wid == 0)
    def _():
        pltpu.sync_copy(t_ref, t_v)
        srcs = []
        for j3 in range(3):
            jj = j3 * 16 + lane
            b = jj // (_K * _K)
            r3 = jj - b * (_K * _K)
            ki = jnp.take_along_axis(
                keep, jnp.minimum(b * _K + r3 // _K, 15), axis=0)
            kj = jnp.take_along_axis(
                keep, jnp.minimum(b * _K + r3 % _K, 15), axis=0)
            src = jnp.minimum(b, _B - 1) * (_N * _N) + ki * _N + kj
            srcs.append(jnp.minimum(src, _B * _N * _N - 1))
        for j in range(_TROWS):
            srcj = jnp.take_along_axis(
                srcs[j // 16], jnp.full((16,), j % 16, jnp.int32), axis=0)
            tbuf[j, :] = plsc.load_gather(t_v, [srcj, lane])
        pltpu.sync_copy(tbuf, outt_ref)


def kernel(feat_map, t_matrix, id_keep, record_len):
    feat2d = feat_map.reshape(_B * _N * _SPLIT, _CL)
    t2d = t_matrix.reshape(_B * _N * _N, 16)
    keep_pad = jnp.zeros((16,), jnp.int32).at[:_B * _K].set(
        id_keep.reshape(-1).astype(jnp.int32))
    rl_pad = jnp.zeros((16,), jnp.int32).at[:_B].set(
        record_len.astype(jnp.int32))
    out_feat, out_t = _sc_gather(feat2d, t2d, keep_pad, rl_pad)
    new_feat_map = out_feat.reshape(_ROWS_OUT, _C, _H, _W)
    new_t_matrix = out_t.reshape(_B, _K, _K, 4, 4)
    new_record_len = jnp.full((_B,), _K, dtype=record_len.dtype)
    return (new_feat_map, new_record_len, new_t_matrix)


# speedup vs baseline: 5.4415x; 5.4415x over previous
import functools

import jax
import jax.numpy as jnp
from jax import lax
from jax.experimental import pallas as pl
from jax.experimental.pallas import tpu as pltpu
from jax.experimental.pallas import tpu_sc as plsc

_B, _N, _K = 4, 5, 3
_C, _H, _W = 256, 96, 96
_CHW = _C * _H * _W
_CL = 3072
_SPLIT = _CHW // _CL
_ROWS_OUT = _B * _K
_TOTAL_CHUNKS = _ROWS_OUT * _SPLIT
_NC, _NS = 2, 16
_NW = _NC * _NS
_PER_W = _TOTAL_CHUNKS // _NW
_GRP = 16
_NGRP = _PER_W // _GRP
_TROWS = _B * _K * _K

_mesh = plsc.VectorSubcoreMesh(
    core_axis_name="c", subcore_axis_name="s", num_cores=_NC, num_subcores=_NS
)


@functools.partial(
    pl.kernel,
    out_type=(
        jax.ShapeDtypeStruct((_TOTAL_CHUNKS, _CL), jnp.float32),
        jax.ShapeDtypeStruct((_TROWS, 16), jnp.float32),
    ),
    mesh=_mesh,
    compiler_params=pltpu.CompilerParams(needs_layout_passes=False),
    scratch_types=[
        pltpu.VMEM((16,), jnp.int32),
        pltpu.VMEM((16,), jnp.int32),
        pltpu.VMEM((16,), jnp.int32),
        pltpu.VMEM((16,), jnp.int32),
        pltpu.VMEM((_NGRP, 16), jnp.int32),
        pltpu.VMEM((_GRP, _CL), jnp.float32),
        pltpu.VMEM((_GRP, _CL), jnp.float32),
        pltpu.VMEM((_B * _N * _N, 16), jnp.float32),
        pltpu.VMEM((_TROWS, 16), jnp.float32),
        pltpu.SemaphoreType.DMA,
        pltpu.SemaphoreType.DMA,
    ],
)
def _sc_gather(feat_ref, t_ref, keep_ref, rl_ref, out_ref, outt_ref,
               keep_v, rl_v, offs_v, grow_v, idxm, buf0, buf1, t_v,
               tbuf, sem_in, sem_out):
    wid = lax.axis_index("s") * _NC + lax.axis_index("c")
    pltpu.sync_copy(keep_ref, keep_v)
    pltpu.sync_copy(rl_ref, rl_v)

    lane = lax.iota(jnp.int32, 16)
    offs = jnp.zeros((16,), jnp.int32)
    for t in range(_B):
        rl_t = plsc.load_gather(rl_v, [jnp.full((16,), t, jnp.int32)])
        offs = offs + jnp.where(lane > t, rl_t, 0)
    offs_v[...] = offs
    keep = keep_v[...]
    grow_v[...] = plsc.load_gather(offs_v, [lane // _K]) + keep

    base = wid * _PER_W
    for g in range(_NGRP):
        cglob = base + g * _GRP + lane
        row = cglob // _SPLIT
        within = cglob - row * _SPLIT
        idxm[g, :] = plsc.load_gather(grow_v, [row]) * _SPLIT + within

    bufs = (buf0, buf1)
    in_dma = [None, None]
    out_dma = [None, None]
    in_dma[0] = pltpu.async_copy(feat_ref.at[idxm.at[0]], bufs[0], sem_in)
    for g in range(_NGRP):
        p = g % 2
        q = 1 - p
        if g + 1 < _NGRP:
            if out_dma[q] is not None:
                out_dma[q].wait()
            in_dma[q] = pltpu.async_copy(
                feat_ref.at[idxm.at[g + 1]], bufs[q], sem_in)
        in_dma[p].wait()
        out_dma[p] = pltpu.async_copy(
            bufs[p], out_ref.at[pl.ds(base + g * _GRP, _GRP)], sem_out)
    for p in range(2):
        if out_dma[p] is not None:
            out_dma[p].wait()

    @pl.when(wid == 0)
    def _():
        pltpu.sync_copy(t_ref, t_v)
        srcs = []
        for j3 in range(3):
            jj = j3 * 16 + lane
            b = jj // (_K * _K)
            r3 = jj - b * (_K * _K)
            ki = plsc.load_gather(keep_v, [jnp.minimum(b * _K + r3 // _K, 15)])
            kj = plsc.load_gather(keep_v, [jnp.minimum(b * _K + r3 % _K, 15)])
            src = jnp.minimum(b, _B - 1) * (_N * _N) + ki * _N + kj
            srcs.append(jnp.minimum(src, _B * _N * _N - 1))
        for j in range(_TROWS):
            srcj = jnp.take_along_axis(
                srcs[j // 16], jnp.full((16,), j % 16, jnp.int32), axis=0)
            tbuf[j, :] = plsc.load_gather(t_v, [srcj, lane])
        pltpu.sync_copy(tbuf, outt_ref)


def kernel(feat_map, t_matrix, id_keep, record_len):
    feat2d = feat_map.reshape(_B * _N * _SPLIT, _CL)
    t2d = t_matrix.reshape(_B * _N * _N, 16)
    keep_pad = jnp.zeros((16,), jnp.int32).at[:_B * _K].set(
        id_keep.reshape(-1).astype(jnp.int32))
    rl_pad = jnp.zeros((16,), jnp.int32).at[:_B].set(
        record_len.astype(jnp.int32))
    out_feat, out_t = _sc_gather(feat2d, t2d, keep_pad, rl_pad)
    new_feat_map = out_feat.reshape(_ROWS_OUT, _C, _H, _W)
    new_t_matrix = out_t.reshape(_B, _K, _K, 4, 4)
    new_record_len = jnp.full((_B,), _K, dtype=record_len.dtype)
    return (new_feat_map, new_record_len, new_t_matrix)


# speedup vs baseline: 10.4679x; 1.9237x over previous
import functools

import jax
import jax.numpy as jnp
from jax import lax
from jax.experimental import pallas as pl
from jax.experimental.pallas import tpu as pltpu
from jax.experimental.pallas import tpu_sc as plsc

_B, _N, _K = 4, 5, 3
_C, _H, _W = 256, 96, 96
_ROWS_OUT = _B * _K
_SLABS_IN = _B * _N * _C
_SLABS_OUT = _ROWS_OUT * _C
_NC, _NS = 2, 16
_NW = _NC * _NS
_PC = 4
_PIECES = _SLABS_OUT // _PC
_PPW = _PIECES // _NW
_PPR = _C // _PC
_TROWS = _B * _K * _K

_mesh = plsc.VectorSubcoreMesh(
    core_axis_name="c", subcore_axis_name="s", num_cores=_NC, num_subcores=_NS
)


@functools.partial(
    pl.kernel,
    out_type=(
        jax.ShapeDtypeStruct((_SLABS_OUT, _H, _W), jnp.float32),
        jax.ShapeDtypeStruct((_TROWS, 16), jnp.float32),
    ),
    mesh=_mesh,
    compiler_params=pltpu.CompilerParams(needs_layout_passes=False),
    scratch_types=[
        pltpu.VMEM((16,), jnp.int32),
        pltpu.VMEM((16,), jnp.int32),
        pltpu.VMEM((_PC, _H, _W), jnp.float32),
        pltpu.VMEM((_PC, _H, _W), jnp.float32),
        pltpu.VMEM((_B * _N * _N, 16), jnp.float32),
        pltpu.VMEM((_TROWS, 16), jnp.float32),
        pltpu.SemaphoreType.DMA,
        pltpu.SemaphoreType.DMA,
    ],
)
def _sc_gather(feat_ref, t_ref, keep_ref, rl_ref, out_ref, outt_ref,
               keep_v, rl_v, buf0, buf1, t_v, tbuf, sem_in, sem_out):
    wid = lax.axis_index("s") * _NC + lax.axis_index("c")
    pltpu.sync_copy(keep_ref, keep_v)
    pltpu.sync_copy(rl_ref, rl_v)

    lane = lax.iota(jnp.int32, 16)
    keep = keep_v[...]
    rl = rl_v[...]
    grow = keep
    for t in range(_B):
        rl_t = jnp.take_along_axis(rl, jnp.full((16,), t, jnp.int32), axis=0)
        grow = grow + jnp.where(lane // _K > t, rl_t, 0)

    def _src_slice(j):
        p = wid * _PPW + j
        r = p // _PPR
        g = jnp.max(jnp.where(lane == r, grow, 0))
        return feat_ref.at[pl.ds(g * _C + (p % _PPR) * _PC, _PC)]

    bufs = (buf0, buf1)
    in_dma = [None, None]
    out_dma = [None, None]
    in_dma[0] = pltpu.async_copy(_src_slice(0), bufs[0], sem_in)
    for j in range(_PPW):
        p = j % 2
        q = 1 - p
        if j + 1 < _PPW:
            if out_dma[q] is not None:
                out_dma[q].wait()
            in_dma[q] = pltpu.async_copy(_src_slice(j + 1), bufs[q], sem_in)
        in_dma[p].wait()
        out_dma[p] = pltpu.async_copy(
            bufs[p],
            out_ref.at[pl.ds((wid * _PPW + j) * _PC, _PC)], sem_out)
    for p in range(2):
        if out_dma[p] is not None:
            out_dma[p].wait()

    @pl.when(wid == 0)
    def _():
        pltpu.sync_copy(t_ref, t_v)
        srcs = []
        for j3 in range(3):
            jj = j3 * 16 + lane
            b = jj // (_K * _K)
            r3 = jj - b * (_K * _K)
            ki = jnp.take_along_axis(
                keep, jnp.minimum(b * _K + r3 // _K, 15), axis=0)
            kj = jnp.take_along_axis(
                keep, jnp.minimum(b * _K + r3 % _K, 15), axis=0)
            src = jnp.minimum(b, _B - 1) * (_N * _N) + ki * _N + kj
            srcs.append(jnp.minimum(src, _B * _N * _N - 1))
        for j in range(_TROWS):
            srcj = jnp.take_along_axis(
                srcs[j // 16], jnp.full((16,), j % 16, jnp.int32), axis=0)
            tbuf[j, :] = plsc.load_gather(t_v, [srcj, lane])
        pltpu.sync_copy(tbuf, outt_ref)


def kernel(feat_map, t_matrix, id_keep, record_len):
    feat3 = feat_map.reshape(_SLABS_IN, _H, _W)
    t2d = t_matrix.reshape(_B * _N * _N, 16)
    keep_pad = jnp.zeros((16,), jnp.int32).at[:_B * _K].set(
        id_keep.reshape(-1).astype(jnp.int32))
    rl_pad = jnp.zeros((16,), jnp.int32).at[:_B].set(
        record_len.astype(jnp.int32))
    out_feat, out_t = _sc_gather(feat3, t2d, keep_pad, rl_pad)
    new_feat_map = out_feat.reshape(_ROWS_OUT, _C, _H, _W)
    new_t_matrix = out_t.reshape(_B, _K, _K, 4, 4)
    new_record_len = jnp.full((_B,), _K, dtype=record_len.dtype)
    return (new_feat_map, new_record_len, new_t_matrix)


# speedup vs baseline: 10.4817x; 1.0013x over previous
import functools

import jax
import jax.numpy as jnp
from jax import lax
from jax.experimental import pallas as pl
from jax.experimental.pallas import tpu as pltpu
from jax.experimental.pallas import tpu_sc as plsc

_B, _N, _K = 4, 5, 3
_C, _H, _W = 256, 96, 96
_ROWS_OUT = _B * _K
_SLABS_IN = _B * _N * _C
_SLABS_OUT = _ROWS_OUT * _C
_NC, _NS = 2, 16
_NW = _NC * _NS
_PC = 4
_PIECES = _SLABS_OUT // _PC
_PPW = _PIECES // _NW
_PPR = _C // _PC
_TROWS = _B * _K * _K

_mesh = plsc.VectorSubcoreMesh(
    core_axis_name="c", subcore_axis_name="s", num_cores=_NC, num_subcores=_NS
)


@functools.partial(
    pl.kernel,
    out_type=(
        jax.ShapeDtypeStruct((_SLABS_OUT, _H, _W), jnp.float32),
        jax.ShapeDtypeStruct((_TROWS, 16), jnp.float32),
    ),
    mesh=_mesh,
    compiler_params=pltpu.CompilerParams(
        needs_layout_passes=False, use_tc_tiling_on_sc=True),
    scratch_types=[
        pltpu.VMEM((16,), jnp.int32),
        pltpu.VMEM((16,), jnp.int32),
        pltpu.VMEM((_PC, _H, _W), jnp.float32),
        pltpu.VMEM((_PC, _H, _W), jnp.float32),
        pltpu.VMEM((_B * _N * _N, 16), jnp.float32),
        pltpu.VMEM((_TROWS, 16), jnp.float32),
        pltpu.SemaphoreType.DMA,
        pltpu.SemaphoreType.DMA,
    ],
)
def _sc_gather(feat_ref, t_ref, keep_ref, rl_ref, out_ref, outt_ref,
               keep_v, rl_v, buf0, buf1, t_v, tbuf, sem_in, sem_out):
    wid = lax.axis_index("s") * _NC + lax.axis_index("c")
    pltpu.sync_copy(keep_ref, keep_v)
    pltpu.sync_copy(rl_ref, rl_v)

    lane = lax.iota(jnp.int32, 16)
    keep = keep_v[...]
    rl = rl_v[...]
    grow = keep
    for t in range(_B):
        rl_t = jnp.take_along_axis(rl, jnp.full((16,), t, jnp.int32), axis=0)
        grow = grow + jnp.where(lane // _K > t, rl_t, 0)

    def _src_slice(j):
        p = wid * _PPW + j
        r = p // _PPR
        g = jnp.max(jnp.where(lane == r, grow, 0))
        return feat_ref.at[pl.ds(g * _C + (p % _PPR) * _PC, _PC)]

    bufs = (buf0, buf1)
    in_dma = [None, None]
    out_dma = [None, None]
    in_dma[0] = pltpu.async_copy(_src_slice(0), bufs[0], sem_in)
    for j in range(_PPW):
        p = j % 2
        q = 1 - p
        if j + 1 < _PPW:
            if out_dma[q] is not None:
                out_dma[q].wait()
            in_dma[q] = pltpu.async_copy(_src_slice(j + 1), bufs[q], sem_in)
        in_dma[p].wait()
        out_dma[p] = pltpu.async_copy(
            bufs[p],
            out_ref.at[pl.ds((wid * _PPW + j) * _PC, _PC)], sem_out)
    for p in range(2):
        if out_dma[p] is not None:
            out_dma[p].wait()

    @pl.when(wid == 0)
    def _():
        pltpu.sync_copy(t_ref, t_v)
        srcs = []
        for j3 in range(3):
            jj = j3 * 16 + lane
            b = jj // (_K * _K)
            r3 = jj - b * (_K * _K)
            ki = jnp.take_along_axis(
                keep, jnp.minimum(b * _K + r3 // _K, 15), axis=0)
            kj = jnp.take_along_axis(
                keep, jnp.minimum(b * _K + r3 % _K, 15), axis=0)
            src = jnp.minimum(b, _B - 1) * (_N * _N) + ki * _N + kj
            srcs.append(jnp.minimum(src, _B * _N * _N - 1))
        for j in range(_TROWS):
            srcj = jnp.take_along_axis(
                srcs[j // 16], jnp.full((16,), j % 16, jnp.int32), axis=0)
            tbuf[j, :] = plsc.load_gather(t_v, [srcj, lane])
        pltpu.sync_copy(tbuf, outt_ref)


def kernel(feat_map, t_matrix, id_keep, record_len):
    feat3 = feat_map.reshape(_SLABS_IN, _H, _W)
    t2d = t_matrix.reshape(_B * _N * _N, 16)
    keep_pad = jnp.zeros((16,), jnp.int32).at[:_B * _K].set(
        id_keep.reshape(-1).astype(jnp.int32))
    rl_pad = jnp.zeros((16,), jnp.int32).at[:_B].set(
        record_len.astype(jnp.int32))
    out_feat, out_t = _sc_gather(feat3, t2d, keep_pad, rl_pad)
    new_feat_map = out_feat.reshape(_ROWS_OUT, _C, _H, _W)
    new_t_matrix = out_t.reshape(_B, _K, _K, 4, 4)
    new_record_len = jnp.full((_B,), _K, dtype=record_len.dtype)
    return (new_feat_map, new_record_len, new_t_matrix)


# speedup vs baseline: 39.6996x; 3.7875x over previous
import functools

import jax
import jax.numpy as jnp
from jax import lax
from jax.experimental import pallas as pl
from jax.experimental.pallas import tpu as pltpu
from jax.experimental.pallas import tpu_sc as plsc

_B, _N, _K = 4, 5, 3
_C, _H, _W = 256, 96, 96
_ROWS_OUT = _B * _K
_RPS = _H * _W
_NC, _NS = 2, 16
_NW = _NC * _NS
_PIECE = 192
_PIECES = _ROWS_OUT * _RPS // _PIECE
_PPW = _PIECES // _NW
_PPR = _RPS // _PIECE
_TROWS = _B * _K * _K

_mesh = plsc.VectorSubcoreMesh(
    core_axis_name="c", subcore_axis_name="s", num_cores=_NC, num_subcores=_NS
)


@functools.partial(
    pl.kernel,
    out_type=(
        jax.ShapeDtypeStruct((_ROWS_OUT * _RPS, _C), jnp.float32),
        jax.ShapeDtypeStruct((_TROWS, 16), jnp.float32),
    ),
    mesh=_mesh,
    compiler_params=pltpu.CompilerParams(needs_layout_passes=False),
    scratch_types=[
        pltpu.VMEM((16,), jnp.int32),
        pltpu.VMEM((16,), jnp.int32),
        pltpu.VMEM((_PIECE, _C), jnp.float32),
        pltpu.VMEM((_PIECE, _C), jnp.float32),
        pltpu.VMEM((_B * _N * _N, 16), jnp.float32),
        pltpu.VMEM((_TROWS, 16), jnp.float32),
        pltpu.SemaphoreType.DMA,
        pltpu.SemaphoreType.DMA,
    ],
)
def _sc_gather(feat_ref, t_ref, keep_ref, rl_ref, out_ref, outt_ref,
               keep_v, rl_v, buf0, buf1, t_v, tbuf, sem_in, sem_out):
    wid = lax.axis_index("s") * _NC + lax.axis_index("c")
    pltpu.sync_copy(keep_ref, keep_v)
    pltpu.sync_copy(rl_ref, rl_v)

    lane = lax.iota(jnp.int32, 16)
    keep = keep_v[...]
    rl = rl_v[...]
    grow = keep
    for t in range(_B):
        rl_t = jnp.take_along_axis(rl, jnp.full((16,), t, jnp.int32), axis=0)
        grow = grow + jnp.where(lane // _K > t, rl_t, 0)

    def _src_slice(j):
        p = wid * _PPW + j
        r = p // _PPR
        g = jnp.max(jnp.where(lane == r, grow, 0))
        return feat_ref.at[pl.ds(g * _RPS + (p % _PPR) * _PIECE, _PIECE)]

    bufs = (buf0, buf1)
    in_dma = [None, None]
    out_dma = [None, None]
    in_dma[0] = pltpu.async_copy(_src_slice(0), bufs[0], sem_in)
    for j in range(_PPW):
        p = j % 2
        q = 1 - p
        if j + 1 < _PPW:
            if out_dma[q] is not None:
                out_dma[q].wait()
            in_dma[q] = pltpu.async_copy(_src_slice(j + 1), bufs[q], sem_in)
        in_dma[p].wait()
        out_dma[p] = pltpu.async_copy(
            bufs[p],
            out_ref.at[pl.ds((wid * _PPW + j) * _PIECE, _PIECE)], sem_out)
    for p in range(2):
        if out_dma[p] is not None:
            out_dma[p].wait()

    @pl.when(wid == 0)
    def _():
        pltpu.sync_copy(t_ref, t_v)
        srcs = []
        for j3 in range(3):
            jj = j3 * 16 + lane
            b = jj // (_K * _K)
            r3 = jj - b * (_K * _K)
            ki = jnp.take_along_axis(
                keep, jnp.minimum(b * _K + r3 // _K, 15), axis=0)
            kj = jnp.take_along_axis(
                keep, jnp.minimum(b * _K + r3 % _K, 15), axis=0)
            src = jnp.minimum(b, _B - 1) * (_N * _N) + ki * _N + kj
            srcs.append(jnp.minimum(src, _B * _N * _N - 1))
        for j in range(_TROWS):
            srcj = jnp.take_along_axis(
                srcs[j // 16], jnp.full((16,), j % 16, jnp.int32), axis=0)
            tbuf[j, :] = plsc.load_gather(t_v, [srcj, lane])
        pltpu.sync_copy(tbuf, outt_ref)


def kernel(feat_map, t_matrix, id_keep, record_len):
    feat2d = feat_map.transpose(0, 2, 3, 1).reshape(_B * _N * _RPS, _C)
    t2d = t_matrix.reshape(_B * _N * _N, 16)
    keep_pad = jnp.zeros((16,), jnp.int32).at[:_B * _K].set(
        id_keep.reshape(-1).astype(jnp.int32))
    rl_pad = jnp.zeros((16,), jnp.int32).at[:_B].set(
        record_len.astype(jnp.int32))
    out_feat, out_t = _sc_gather(feat2d, t2d, keep_pad, rl_pad)
    new_feat_map = out_feat.reshape(_ROWS_OUT, _H, _W, _C).transpose(0, 3, 1, 2)
    new_t_matrix = out_t.reshape(_B, _K, _K, 4, 4)
    new_record_len = jnp.full((_B,), _K, dtype=record_len.dtype)
    return (new_feat_map, new_record_len, new_t_matrix)


# speedup vs baseline: 39.9540x; 1.0064x over previous
import functools

import jax
import jax.numpy as jnp
from jax import lax
from jax.experimental import pallas as pl
from jax.experimental.pallas import tpu as pltpu
from jax.experimental.pallas import tpu_sc as plsc

_B, _N, _K = 4, 5, 3
_C, _H, _W = 256, 96, 96
_ROWS_OUT = _B * _K
_RPS = _H * _W
_NC, _NS = 2, 16
_NW = _NC * _NS
_PIECE = 144
_NBUF = 3
_PIECES = _ROWS_OUT * _RPS // _PIECE
_PPW = _PIECES // _NW
_PPR = _RPS // _PIECE
_TROWS = _B * _K * _K

_mesh = plsc.VectorSubcoreMesh(
    core_axis_name="c", subcore_axis_name="s", num_cores=_NC, num_subcores=_NS
)


@functools.partial(
    pl.kernel,
    out_type=(
        jax.ShapeDtypeStruct((_ROWS_OUT * _RPS, _C), jnp.float32),
        jax.ShapeDtypeStruct((_TROWS, 16), jnp.float32),
    ),
    mesh=_mesh,
    compiler_params=pltpu.CompilerParams(needs_layout_passes=False),
    scratch_types=[
        pltpu.VMEM((16,), jnp.int32),
        pltpu.VMEM((16,), jnp.int32),
        pltpu.VMEM((_PIECE, _C), jnp.float32),
        pltpu.VMEM((_PIECE, _C), jnp.float32),
        pltpu.VMEM((_PIECE, _C), jnp.float32),
        pltpu.VMEM((_B * _N * _N, 16), jnp.float32),
        pltpu.VMEM((_TROWS, 16), jnp.float32),
        pltpu.SemaphoreType.DMA,
        pltpu.SemaphoreType.DMA,
    ],
)
def _sc_gather(feat_ref, t_ref, keep_ref, rl_ref, out_ref, outt_ref,
               keep_v, rl_v, buf0, buf1, buf2, t_v, tbuf, sem_in, sem_out):
    wid = lax.axis_index("s") * _NC + lax.axis_index("c")
    pltpu.sync_copy(keep_ref, keep_v)
    pltpu.sync_copy(rl_ref, rl_v)

    lane = lax.iota(jnp.int32, 16)
    keep = keep_v[...]
    rl = rl_v[...]
    grow = keep
    for t in range(_B):
        rl_t = jnp.take_along_axis(rl, jnp.full((16,), t, jnp.int32), axis=0)
        grow = grow + jnp.where(lane // _K > t, rl_t, 0)

    def _src_slice(j):
        p = wid * _PPW + j
        r = p // _PPR
        g = jnp.max(jnp.where(lane == r, grow, 0))
        return feat_ref.at[pl.ds(g * _RPS + (p % _PPR) * _PIECE, _PIECE)]

    bufs = (buf0, buf1, buf2)
    in_dma = [None] * _NBUF
    out_dma = [None] * _NBUF
    for b in range(_NBUF - 1):
        in_dma[b] = pltpu.async_copy(_src_slice(b), bufs[b], sem_in)
    for j in range(_PPW):
        p = j % _NBUF
        q = (j + _NBUF - 1) % _NBUF
        if j + _NBUF - 1 < _PPW:
            if out_dma[q] is not None:
                out_dma[q].wait()
            in_dma[q] = pltpu.async_copy(
                _src_slice(j + _NBUF - 1), bufs[q], sem_in)
        in_dma[p].wait()
        out_dma[p] = pltpu.async_copy(
            bufs[p],
            out_ref.at[pl.ds((wid * _PPW + j) * _PIECE, _PIECE)], sem_out)
    for p in range(_NBUF):
        if out_dma[p] is not None:
            out_dma[p].wait()

    @pl.when(wid == 0)
    def _():
        pltpu.sync_copy(t_ref, t_v)
        srcs = []
        for j3 in range(3):
            jj = j3 * 16 + lane
            b = jj // (_K * _K)
            r3 = jj - b * (_K * _K)
            ki = jnp.take_along_axis(
                keep, jnp.minimum(b * _K + r3 // _K, 15), axis=0)
            kj = jnp.take_along_axis(
                keep, jnp.minimum(b * _K + r3 % _K, 15), axis=0)
            src = jnp.minimum(b, _B - 1) * (_N * _N) + ki * _N + kj
            srcs.append(jnp.minimum(src, _B * _N * _N - 1))
        for j in range(_TROWS):
            srcj = jnp.take_along_axis(
                srcs[j // 16], jnp.full((16,), j % 16, jnp.int32), axis=0)
            tbuf[j, :] = plsc.load_gather(t_v, [srcj, lane])
        pltpu.sync_copy(tbuf, outt_ref)


def kernel(feat_map, t_matrix, id_keep, record_len):
    feat2d = feat_map.transpose(0, 2, 3, 1).reshape(_B * _N * _RPS, _C)
    t2d = t_matrix.reshape(_B * _N * _N, 16)
    keep_pad = jnp.zeros((16,), jnp.int32).at[:_B * _K].set(
        id_keep.reshape(-1).astype(jnp.int32))
    rl_pad = jnp.zeros((16,), jnp.int32).at[:_B].set(
        record_len.astype(jnp.int32))
    out_feat, out_t = _sc_gather(feat2d, t2d, keep_pad, rl_pad)
    new_feat_map = out_feat.reshape(_ROWS_OUT, _H, _W, _C).transpose(0, 3, 1, 2)
    new_t_matrix = out_t.reshape(_B, _K, _K, 4, 4)
    new_record_len = jnp.full((_B,), _K, dtype=record_len.dtype)
    return (new_feat_map, new_record_len, new_t_matrix)
